# baseline (speedup 1.0000x reference)
import functools

import jax
import jax.numpy as jnp
from jax import lax
from jax.experimental import pallas as pl
from jax.experimental.pallas import tpu as pltpu
from jax.experimental.pallas import tpu_sc as plsc

_THRESH = 0.7
_MIN_KEPT = 100000
_NCLS = 96
_BH = 32

_B = 4
_N = 147456
_NSUB = 16
_CHUNK = _N // _NSUB
_NVEC = _CHUNK // 16


def _loss_kernel(logits_ref, targets_ref, loss_ref, pres_ref):
    h = pl.program_id(1)
    x = logits_ref[...]
    t = targets_ref[...]
    m = jnp.max(x, axis=1, keepdims=True)
    e = jnp.exp(x - m)
    s = jnp.sum(e, axis=1)
    cls = jax.lax.broadcasted_iota(jnp.int32, x.shape, 1)
    oh = cls == t[:, None, :, :]
    et = jnp.sum(jnp.where(oh, e, 0.0), axis=1)
    p = et / s
    loss = -jnp.log(p + 1e-7)
    loss_ref[...] = loss
    ph = jnp.max(jnp.where(oh, 1.0, 0.0), axis=(2, 3))
    ph = jnp.pad(ph, ((0, 0), (0, 128 - _NCLS)))[:, None, :]

    @pl.when(h == 0)
    def _():
        pres_ref[...] = ph

    @pl.when(h != 0)
    def _():
        pres_ref[...] = jnp.maximum(pres_ref[...], ph)


def _sc_select_kernel(loss_hbm, pres_hbm, out_hbm, slots_hbm,
                      loss_v, cand_a, cand_b, hist_v, comb_v,
                      big_v, pres_v, *, rank):
    mask_v = loss_v
    cid = lax.axis_index("c")
    sid = lax.axis_index("s")
    wid = cid * 16 + sid
    base = sid * _CHUNK
    lane = lax.iota(jnp.int32, 16)
    ones = jnp.ones((16,), jnp.float32)
    zeros16 = jnp.zeros((16,), jnp.float32)

    def combine_tiles(parity):
        pltpu.sync_copy(comb_v, slots_hbm.at[parity * 32 + wid])
        plsc.subcore_barrier()
        for half in range(2):
            pltpu.sync_copy(
                slots_hbm.at[pl.ds(parity * 32 + cid * 16 + half * 8, 8)],
                big_v)
            for jj in range(2):
                for v in range(8):
                    acc = big_v[0, jj, v, :]
                    for w in range(1, 8):
                        acc = acc + big_v[w, jj, v, :]
                    if half == 0:
                        comb_v[jj, v, :] = acc
                    else:
                        comb_v[jj, v, :] = comb_v[jj, v, :] + acc

    def zero_hist():
        def zh(i, _):
            for k in range(4):
                hist_v[pl.ds(i * 64 + k * 16, 16)] = zeros16
            return 0
        lax.fori_loop(0, 64, zh, 0)

    def lane_combine():
        for jj in range(2):
            for v in range(8):
                acc = zeros16
                for l in range(16):
                    acc = acc + hist_v[pl.ds(jj * 2048 + l * 128 + v * 16,
                                             16)]
                comb_v[jj, v, :] = acc

    def level_scan(jj, r_rem):
        bsel = jnp.int32(-1)
        below = jnp.float32(0.0)
        carry = jnp.float32(0.0)
        for v in range(8):
            vec = comb_v[jj, v, :]
            incl = plsc.cumsum(vec)
            pred = (carry + incl) >= r_rem
            lmin = jnp.min(jnp.where(pred, lane, 16))
            bel_v = carry + jnp.sum(jnp.where(lane < lmin, vec, 0.0))
            hit = jnp.logical_and(bsel < 0, lmin < 16)
            bsel = jnp.where(hit, v * 16 + lmin, bsel)
            below = jnp.where(hit, bel_v, below)
            carry = carry + jnp.sum(vec)
        return bsel, r_rem - below

    for j in range(2):
        b = 2 * cid + j
        pltpu.sync_copy(loss_hbm.at[b, pl.ds(base, _CHUNK)],
                        loss_v.at[pl.ds(j * _CHUNK, _CHUNK)])
        pltpu.sync_copy(pres_hbm.at[b], pres_v.at[pl.ds(j * 128, 128)])

    zero_hist()

    def keys_of(jj, off):
        x = loss_v[pl.ds(jj * _CHUNK + off, 16)]
        ib = plsc.bitcast(x, jnp.int32)
        ub = plsc.bitcast(x, jnp.uint32)
        return jnp.where(ib >= 0, ub | jnp.uint32(0x80000000), ~ub)

    def key_sweep(i, _):
        for k in range(4):
            off = i * 64 + k * 16
            for jj in range(2):
                kv = keys_of(jj, off)
                dig = lax.shift_right_logical(kv,
                                              jnp.uint32(25)).astype(jnp.int32)
                plsc.addupdate_scatter(hist_v,
                                       [jj * 2048 + lane * 128 + dig], ones)
        return 0
    lax.fori_loop(0, _NVEC // 4, key_sweep, 0)
    lane_combine()
    combine_tiles(0)

    bsel = [None, None]
    r_rem = [None, None]
    kstar = [None, None]
    nn = [None, None]
    for jj in range(2):
        b1, rr = level_scan(jj, jnp.float32(rank))
        bsel[jj] = b1
        r_rem[jj] = rr
        kstar[jj] = lax.shift_left(b1.astype(jnp.uint32), jnp.uint32(25))

        def compact1(i, off):
            for k in range(4):
                kv = keys_of(jj, i * 64 + k * 16)
                kr = plsc.bitcast(kv, jnp.int32)
                dig = lax.shift_right_logical(kv,
                                              jnp.uint32(25)).astype(jnp.int32)
                m = dig == b1
                mi = m.astype(jnp.int32)
                pos = off + plsc.cumsum(mi) - mi
                plsc.store_scatter(cand_a, [jj * (_CHUNK + 16) + pos], kr,
                                   mask=m)
                off = off + jnp.sum(mi)
            return off
        nn[jj] = lax.fori_loop(0, _NVEC // 4, compact1, jnp.int32(0))

    specs = [(cand_a, cand_b, 18, 0x7F),
             (cand_b, cand_a, 11, 0x7F),
             (cand_a, cand_b, 4, 0x7F),
             (cand_b, None, 0, 0xF)]
    for lvl, (srcb, dstb, shift, dmask) in enumerate(specs):
        zero_hist()
        for jj in range(2):
            def sweep(i, _):
                kv = plsc.bitcast(
                    srcb[pl.ds(jj * (_CHUNK + 16) + i * 16, 16)], jnp.uint32)
                valid = (i * 16 + lane) < nn[jj]
                dig = (lax.shift_right_logical(kv, jnp.uint32(shift))
                       & jnp.uint32(dmask)).astype(jnp.int32)
                plsc.addupdate_scatter(hist_v,
                                       [jj * 2048 + lane * 128 + dig],
                                       ones, mask=valid)
                return 0
            lax.fori_loop(0, (nn[jj] + 15) // 16, sweep, 0)
        lane_combine()
        combine_tiles((lvl + 1) % 2)
        for jj in range(2):
            bl, rr = level_scan(jj, r_rem[jj])
            r_rem[jj] = rr
            kstar[jj] = kstar[jj] | lax.shift_left(bl.astype(jnp.uint32),
                                                   jnp.uint32(shift))
            if dstb is not None:
                def cbody(i, off):
                    kr = srcb[pl.ds(jj * (_CHUNK + 16) + i * 16, 16)]
                    kv = plsc.bitcast(kr, jnp.uint32)
                    valid = (i * 16 + lane) < nn[jj]
                    dig = (lax.shift_right_logical(kv, jnp.uint32(shift))
                           & jnp.uint32(dmask)).astype(jnp.int32)
                    m = jnp.logical_and(valid, dig == bl)
                    mi = m.astype(jnp.int32)
                    pos = off + plsc.cumsum(mi) - mi
                    plsc.store_scatter(dstb, [jj * (_CHUNK + 16) + pos], kr,
                                       mask=m)
                    return off + jnp.sum(mi)
                nn[jj] = lax.fori_loop(0, (nn[jj] + 15) // 16, cbody,
                                       jnp.int32(0))

    def mask_sweep(i, _):
        for k in range(4):
            off = i * 64 + k * 16
            for jj in range(2):
                kv = keys_of(jj, off)
                mask_v[pl.ds(jj * _CHUNK + off, 16)] = (
                    kv > kstar[jj]).astype(jnp.float32)
        return 0
    lax.fori_loop(0, _NVEC // 4, mask_sweep, 0)

    @pl.when(sid == 0)
    def _():
        for jj in range(2):
            for v in range(6):
                slm = pl.ds(jj * _CHUNK + v * 16, 16)
                slp = pl.ds(jj * 128 + v * 16, 16)
                mask_v[slm] = jnp.maximum(mask_v[slm], pres_v[slp])

    for j in range(2):
        b = 2 * cid + j
        pltpu.sync_copy(mask_v.at[pl.ds(j * _CHUNK, _CHUNK)],
                        out_hbm.at[b, pl.ds(base, _CHUNK)])


def kernel(logits, targets):
    b, c, hh, ww = logits.shape
    n = hh * ww
    targets = targets.astype(jnp.int32)

    grid1 = (b, hh // _BH)
    loss, pres = pl.pallas_call(
        _loss_kernel,
        grid=grid1,
        in_specs=[
            pl.BlockSpec((1, c, _BH, ww), lambda i, j: (i, 0, j, 0)),
            pl.BlockSpec((1, _BH, ww), lambda i, j: (i, j, 0)),
        ],
        out_specs=[
            pl.BlockSpec((1, _BH, ww), lambda i, j: (i, j, 0)),
            pl.BlockSpec((1, 1, 128), lambda i, j: (i, 0, 0)),
        ],
        out_shape=[
            jax.ShapeDtypeStruct((b, hh, ww), jnp.float32),
            jax.ShapeDtypeStruct((b, 1, 128), jnp.float32),
        ],
        compiler_params=pltpu.CompilerParams(
            dimension_semantics=("parallel", "arbitrary"),
        ),
    )(logits, targets)

    rank = min(max(_MIN_KEPT, int(n * _THRESH)), n - 1) + 1
    mesh = plsc.VectorSubcoreMesh(core_axis_name="c", subcore_axis_name="s")
    mask = pl.kernel(
        functools.partial(_sc_select_kernel, rank=rank),
        out_type=[jax.ShapeDtypeStruct((b, n), jnp.float32),
                  jax.ShapeDtypeStruct((64, 2, 16, 16), jnp.float32)],
        mesh=mesh,
        compiler_params=pltpu.CompilerParams(needs_layout_passes=False),
        scratch_types=[
            pltpu.VMEM((2 * _CHUNK,), jnp.float32),
            pltpu.VMEM((2 * (_CHUNK + 16),), jnp.int32),
            pltpu.VMEM((2 * (_CHUNK + 16),), jnp.int32),
            pltpu.VMEM((4096,), jnp.float32),
            pltpu.VMEM((2, 16, 16), jnp.float32),
            pltpu.VMEM((8, 2, 16, 16), jnp.float32),
            pltpu.VMEM((256,), jnp.float32),
        ],
    )(loss.reshape(b, n), pres.reshape(b, 128))[0]
    return mask.reshape(b, hh, ww)


# speedup vs baseline: 1.0607x; 1.0607x over previous
import functools

import jax
import jax.numpy as jnp
from jax import lax
from jax.experimental import pallas as pl
from jax.experimental.pallas import tpu as pltpu
from jax.experimental.pallas import tpu_sc as plsc

_THRESH = 0.7
_MIN_KEPT = 100000
_NCLS = 96
_BH = 32

_B = 4
_N = 147456
_NSUB = 16
_CHUNK = _N // _NSUB
_NVEC = _CHUNK // 16


def _loss_kernel(logits_ref, targets_ref, loss_ref, pres_ref):
    h = pl.program_id(1)
    x = logits_ref[...]
    t = targets_ref[...]
    m = jnp.max(x, axis=1, keepdims=True)
    e = jnp.exp(x - m)
    s = jnp.sum(e, axis=1)
    cls = jax.lax.broadcasted_iota(jnp.int32, x.shape, 1)
    oh = cls == t[:, None, :, :]
    et = jnp.sum(jnp.where(oh, e, 0.0), axis=1)
    p = et / s
    loss = -jnp.log(p + 1e-7)
    loss_ref[...] = loss
    ph = jnp.max(jnp.where(oh, 1.0, 0.0), axis=(2, 3))
    ph = jnp.pad(ph, ((0, 0), (0, 128 - _NCLS)))[:, None, :]

    @pl.when(h == 0)
    def _():
        pres_ref[...] = ph

    @pl.when(h != 0)
    def _():
        pres_ref[...] = jnp.maximum(pres_ref[...], ph)


def _sc_select_kernel(loss_hbm, pres_hbm, out_hbm, slots_hbm,
                      loss_v, cand_a, hist_v, comb_v,
                      big_v, pres_v, *, rank):
    mask_v = loss_v
    cid = lax.axis_index("c")
    sid = lax.axis_index("s")
    wid = cid * 16 + sid
    base = sid * _CHUNK
    lane = lax.iota(jnp.int32, 16)
    ones = jnp.ones((16,), jnp.float32)
    zeros16 = jnp.zeros((16,), jnp.float32)

    def combine_tiles(parity):
        pltpu.sync_copy(comb_v, slots_hbm.at[parity * 32 + wid])
        plsc.subcore_barrier()
        for half in range(2):
            pltpu.sync_copy(
                slots_hbm.at[pl.ds(parity * 32 + cid * 16 + half * 8, 8)],
                big_v)
            for jj in range(2):
                for v in range(8):
                    acc = big_v[0, jj, v, :]
                    for w in range(1, 8):
                        acc = acc + big_v[w, jj, v, :]
                    if half == 0:
                        comb_v[jj, v, :] = acc
                    else:
                        comb_v[jj, v, :] = comb_v[jj, v, :] + acc

    def zero_hist():
        def zh(i, _):
            for k in range(4):
                hist_v[pl.ds(i * 64 + k * 16, 16)] = zeros16
            return 0
        lax.fori_loop(0, 64, zh, 0)

    def lane_combine():
        for jj in range(2):
            for v in range(8):
                acc = zeros16
                for l in range(16):
                    acc = acc + hist_v[pl.ds(jj * 2048 + l * 128 + v * 16,
                                             16)]
                comb_v[jj, v, :] = acc

    def level_scan(jj, r_rem):
        bsel = jnp.int32(-1)
        below = jnp.float32(0.0)
        carry = jnp.float32(0.0)
        for v in range(8):
            vec = comb_v[jj, v, :]
            incl = plsc.cumsum(vec)
            pred = (carry + incl) >= r_rem
            lmin = jnp.min(jnp.where(pred, lane, 16))
            bel_v = carry + jnp.sum(jnp.where(lane < lmin, vec, 0.0))
            hit = jnp.logical_and(bsel < 0, lmin < 16)
            bsel = jnp.where(hit, v * 16 + lmin, bsel)
            below = jnp.where(hit, bel_v, below)
            carry = carry + jnp.sum(vec)
        return bsel, r_rem - below

    for j in range(2):
        b = 2 * cid + j
        pltpu.sync_copy(loss_hbm.at[b, pl.ds(base, _CHUNK)],
                        loss_v.at[pl.ds(j * _CHUNK, _CHUNK)])
        pltpu.sync_copy(pres_hbm.at[b], pres_v.at[pl.ds(j * 128, 128)])

    def keys_of(jj, off):
        x = loss_v[pl.ds(jj * _CHUNK + off, 16)]
        ib = plsc.bitcast(x, jnp.int32)
        ub = plsc.bitcast(x, jnp.uint32)
        return jnp.where(ib >= 0, ub | jnp.uint32(0x80000000), ~ub)

    zero_hist()

    def key_sweep(i, _):
        for k in range(4):
            off = i * 64 + k * 16
            for jj in range(2):
                kv = keys_of(jj, off)
                dig = lax.shift_right_logical(kv,
                                              jnp.uint32(25)).astype(jnp.int32)
                plsc.addupdate_scatter(hist_v,
                                       [jj * 2048 + lane * 128 + dig], ones)
        return 0
    lax.fori_loop(0, _NVEC // 4, key_sweep, 0)
    lane_combine()
    combine_tiles(0)

    bsel1 = [None, None]
    bsel2 = [None, None]
    r_rem = [None, None]
    kstar = [None, None]
    nn = [None, None]
    for jj in range(2):
        bsel1[jj], r_rem[jj] = level_scan(jj, jnp.float32(rank))

    zero_hist()

    def l2_sweep(i, _):
        for k in range(4):
            off = i * 64 + k * 16
            for jj in range(2):
                kv = keys_of(jj, off)
                d1 = lax.shift_right_logical(kv,
                                             jnp.uint32(25)).astype(jnp.int32)
                d2 = (lax.shift_right_logical(kv, jnp.uint32(18))
                      & jnp.uint32(0x7F)).astype(jnp.int32)
                plsc.addupdate_scatter(hist_v,
                                       [jj * 2048 + lane * 128 + d2],
                                       ones, mask=d1 == bsel1[jj])
        return 0
    lax.fori_loop(0, _NVEC // 4, l2_sweep, 0)
    lane_combine()
    combine_tiles(1)
    for jj in range(2):
        bsel2[jj], r_rem[jj] = level_scan(jj, r_rem[jj])

    zero_hist()
    for jj in range(2):
        p14 = (bsel1[jj] * 128 + bsel2[jj]).astype(jnp.uint32)

        def l3_sweep(i, off):
            for k in range(4):
                kv = keys_of(jj, i * 64 + k * 16)
                m = lax.shift_right_logical(kv, jnp.uint32(18)) == p14
                d3 = (lax.shift_right_logical(kv, jnp.uint32(11))
                      & jnp.uint32(0x7F)).astype(jnp.int32)
                plsc.addupdate_scatter(hist_v,
                                       [jj * 2048 + lane * 128 + d3],
                                       ones, mask=m)
                mi = m.astype(jnp.int32)
                pos = off + plsc.cumsum(mi) - mi
                plsc.store_scatter(cand_a, [jj * (_CHUNK + 16) + pos],
                                   plsc.bitcast(kv, jnp.int32), mask=m)
                off = off + jnp.sum(mi)
            return off
        nn[jj] = lax.fori_loop(0, _NVEC // 4, l3_sweep, jnp.int32(0))
    lane_combine()
    combine_tiles(0)

    bsel3 = [None, None]
    bsel4 = [None, None]
    bsel5 = [None, None]
    for jj in range(2):
        bsel3[jj], r_rem[jj] = level_scan(jj, r_rem[jj])

    zero_hist()
    for jj in range(2):
        def l4_sweep(i, _):
            kr = cand_a[pl.ds(jj * (_CHUNK + 16) + i * 16, 16)]
            kv = plsc.bitcast(kr, jnp.uint32)
            valid = (i * 16 + lane) < nn[jj]
            d3 = (lax.shift_right_logical(kv, jnp.uint32(11))
                  & jnp.uint32(0x7F)).astype(jnp.int32)
            d4 = (lax.shift_right_logical(kv, jnp.uint32(4))
                  & jnp.uint32(0x7F)).astype(jnp.int32)
            m = jnp.logical_and(valid, d3 == bsel3[jj])
            plsc.addupdate_scatter(hist_v, [jj * 2048 + lane * 128 + d4],
                                   ones, mask=m)
            return 0
        lax.fori_loop(0, (nn[jj] + 15) // 16, l4_sweep, 0)
    lane_combine()
    combine_tiles(1)
    for jj in range(2):
        bsel4[jj], r_rem[jj] = level_scan(jj, r_rem[jj])

    zero_hist()
    for jj in range(2):
        def l5_sweep(i, _):
            kr = cand_a[pl.ds(jj * (_CHUNK + 16) + i * 16, 16)]
            kv = plsc.bitcast(kr, jnp.uint32)
            valid = (i * 16 + lane) < nn[jj]
            d3 = (lax.shift_right_logical(kv, jnp.uint32(11))
                  & jnp.uint32(0x7F)).astype(jnp.int32)
            d4 = (lax.shift_right_logical(kv, jnp.uint32(4))
                  & jnp.uint32(0x7F)).astype(jnp.int32)
            d5 = (kv & jnp.uint32(0xF)).astype(jnp.int32)
            m = jnp.logical_and(valid,
                                jnp.logical_and(d3 == bsel3[jj],
                                                d4 == bsel4[jj]))
            plsc.addupdate_scatter(hist_v, [jj * 2048 + lane * 128 + d5],
                                   ones, mask=m)
            return 0
        lax.fori_loop(0, (nn[jj] + 15) // 16, l5_sweep, 0)
    lane_combine()
    combine_tiles(0)
    for jj in range(2):
        bsel5[jj], r_rem[jj] = level_scan(jj, r_rem[jj])
        kstar[jj] = (lax.shift_left(bsel1[jj].astype(jnp.uint32),
                                    jnp.uint32(25))
                     | lax.shift_left(bsel2[jj].astype(jnp.uint32),
                                      jnp.uint32(18))
                     | lax.shift_left(bsel3[jj].astype(jnp.uint32),
                                      jnp.uint32(11))
                     | lax.shift_left(bsel4[jj].astype(jnp.uint32),
                                      jnp.uint32(4))
                     | bsel5[jj].astype(jnp.uint32))

    def mask_sweep(i, _):
        for k in range(4):
            off = i * 64 + k * 16
            for jj in range(2):
                kv = keys_of(jj, off)
                mask_v[pl.ds(jj * _CHUNK + off, 16)] = (
                    kv > kstar[jj]).astype(jnp.float32)
        return 0
    lax.fori_loop(0, _NVEC // 4, mask_sweep, 0)

    @pl.when(sid == 0)
    def _():
        for jj in range(2):
            for v in range(6):
                slm = pl.ds(jj * _CHUNK + v * 16, 16)
                slp = pl.ds(jj * 128 + v * 16, 16)
                mask_v[slm] = jnp.maximum(mask_v[slm], pres_v[slp])

    for j in range(2):
        b = 2 * cid + j
        pltpu.sync_copy(mask_v.at[pl.ds(j * _CHUNK, _CHUNK)],
                        out_hbm.at[b, pl.ds(base, _CHUNK)])


def kernel(logits, targets):
    b, c, hh, ww = logits.shape
    n = hh * ww
    targets = targets.astype(jnp.int32)

    grid1 = (b, hh // _BH)
    loss, pres = pl.pallas_call(
        _loss_kernel,
        grid=grid1,
        in_specs=[
            pl.BlockSpec((1, c, _BH, ww), lambda i, j: (i, 0, j, 0)),
            pl.BlockSpec((1, _BH, ww), lambda i, j: (i, j, 0)),
        ],
        out_specs=[
            pl.BlockSpec((1, _BH, ww), lambda i, j: (i, j, 0)),
            pl.BlockSpec((1, 1, 128), lambda i, j: (i, 0, 0)),
        ],
        out_shape=[
            jax.ShapeDtypeStruct((b, hh, ww), jnp.float32),
            jax.ShapeDtypeStruct((b, 1, 128), jnp.float32),
        ],
        compiler_params=pltpu.CompilerParams(
            dimension_semantics=("parallel", "arbitrary"),
        ),
    )(logits, targets)

    rank = min(max(_MIN_KEPT, int(n * _THRESH)), n - 1) + 1
    mesh = plsc.VectorSubcoreMesh(core_axis_name="c", subcore_axis_name="s")
    mask = pl.kernel(
        functools.partial(_sc_select_kernel, rank=rank),
        out_type=[jax.ShapeDtypeStruct((b, n), jnp.float32),
                  jax.ShapeDtypeStruct((64, 2, 16, 16), jnp.float32)],
        mesh=mesh,
        compiler_params=pltpu.CompilerParams(needs_layout_passes=False),
        scratch_types=[
            pltpu.VMEM((2 * _CHUNK,), jnp.float32),
            pltpu.VMEM((2 * (_CHUNK + 16),), jnp.int32),
            pltpu.VMEM((4096,), jnp.float32),
            pltpu.VMEM((2, 16, 16), jnp.float32),
            pltpu.VMEM((8, 2, 16, 16), jnp.float32),
            pltpu.VMEM((256,), jnp.float32),
        ],
    )(loss.reshape(b, n), pres.reshape(b, 128))[0]
    return mask.reshape(b, hh, ww)
